# baseline (speedup 1.0000x reference)
import functools

import jax
import jax.numpy as jnp
from jax import lax
from jax.experimental import pallas as pl
from jax.experimental.pallas import tpu as pltpu
from jax.experimental.pallas import tpu_sc as plsc

_B, _N, _D = 128, 64, 128
_NC, _NS, _L = 2, 16, 16
_NW = _NC * _NS
_BPW = _B // _NW
_NG = _D // _L
_INV = 1.0 / (_N - 1)


@functools.partial(
    pl.kernel,
    out_type=jax.ShapeDtypeStruct((_B, _N, 2 * _D), jnp.float32),
    mesh=plsc.VectorSubcoreMesh(core_axis_name="c", subcore_axis_name="s"),
    scratch_types=[
        pltpu.VMEM((_N, _D), jnp.float32),
        pltpu.VMEM((_N, _D), jnp.float32),
        pltpu.SemaphoreType.DMA,
    ],
)
def _sc_encode(x_hbm, out_hbm, x_v, o_v, sem):
    wid = lax.axis_index("s") * _NC + lax.axis_index("c")
    for bi in range(_BPW):
        b = wid * _BPW + bi
        cp = pltpu.async_copy(x_hbm.at[b], out_hbm.at[b, :, pl.ds(_D, _D)], sem)
        pltpu.sync_copy(x_hbm.at[b], x_v)

        def _acc(n, accs):
            return tuple(accs[g] + x_v[n, pl.ds(_L * g, _L)] for g in range(_NG))

        accs = lax.fori_loop(
            0, _N, _acc, tuple(jnp.zeros((_L,), jnp.float32) for _ in range(_NG))
        )

        def _emit(n, c):
            for g in range(_NG):
                xv = x_v[n, pl.ds(_L * g, _L)]
                o_v[n, pl.ds(_L * g, _L)] = (accs[g] - xv) * _INV
            return c

        lax.fori_loop(0, _N, _emit, 0)
        pltpu.sync_copy(o_v, out_hbm.at[b, :, pl.ds(0, _D)])
        cp.wait()


def kernel(inputs, send_edges, recv_edges, edge2node_mat):
    return _sc_encode(inputs)


# speedup vs baseline: 3.4655x; 3.4655x over previous
import functools

import jax
import jax.numpy as jnp
from jax import lax
from jax.experimental import pallas as pl
from jax.experimental.pallas import tpu as pltpu
from jax.experimental.pallas import tpu_sc as plsc

_B, _N, _D = 128, 64, 128
_NC, _NS, _L = 2, 16, 16
_NW = _NC * _NS
_BPW = _B // _NW
_NG = _D // _L
_INV = 1.0 / (_N - 1)


@functools.partial(
    pl.kernel,
    out_type=jax.ShapeDtypeStruct((_B, _N, 2 * _D), jnp.float32),
    mesh=plsc.VectorSubcoreMesh(core_axis_name="c", subcore_axis_name="s"),
    scratch_types=[
        pltpu.VMEM((_N, _D), jnp.float32),
        pltpu.VMEM((_N, 2 * _D), jnp.float32),
    ],
)
def _sc_encode(x_hbm, out_hbm, x_v, o_v):
    wid = lax.axis_index("s") * _NC + lax.axis_index("c")
    for bi in range(_BPW):
        b = wid * _BPW + bi
        pltpu.sync_copy(x_hbm.at[b], x_v)

        def _acc(n, accs):
            return tuple(accs[g] + x_v[n, pl.ds(_L * g, _L)] for g in range(_NG))

        accs = lax.fori_loop(
            0, _N, _acc, tuple(jnp.zeros((_L,), jnp.float32) for _ in range(_NG)),
            unroll=4,
        )

        def _emit(n, c):
            for g in range(_NG):
                xv = x_v[n, pl.ds(_L * g, _L)]
                o_v[n, pl.ds(_L * g, _L)] = (accs[g] - xv) * _INV
                o_v[n, pl.ds(_D + _L * g, _L)] = xv
            return c

        lax.fori_loop(0, _N, _emit, 0, unroll=4)
        pltpu.sync_copy(o_v, out_hbm.at[b])


def kernel(inputs, send_edges, recv_edges, edge2node_mat):
    return _sc_encode(inputs)


# speedup vs baseline: 3.9527x; 1.1406x over previous
import functools

import jax
import jax.numpy as jnp
from jax import lax
from jax.experimental import pallas as pl
from jax.experimental.pallas import tpu as pltpu
from jax.experimental.pallas import tpu_sc as plsc

_B, _N, _D = 128, 64, 128
_NC, _NS, _L = 2, 16, 16
_NW = _NC * _NS
_BPW = _B // _NW
_NG = _D // _L
_INV = 1.0 / (_N - 1)


def _compute(x_v, o_v):
    def _acc(n, accs):
        return tuple(accs[g] + x_v[n, pl.ds(_L * g, _L)] for g in range(_NG))

    accs = lax.fori_loop(
        0, _N, _acc, tuple(jnp.zeros((_L,), jnp.float32) for _ in range(_NG)),
        unroll=4,
    )

    def _emit(n, c):
        for g in range(_NG):
            xv = x_v[n, pl.ds(_L * g, _L)]
            o_v[n, pl.ds(_L * g, _L)] = (accs[g] - xv) * _INV
            o_v[n, pl.ds(_D + _L * g, _L)] = xv
        return c

    lax.fori_loop(0, _N, _emit, 0, unroll=4)


@functools.partial(
    pl.kernel,
    out_type=jax.ShapeDtypeStruct((_B, _N, 2 * _D), jnp.float32),
    mesh=plsc.VectorSubcoreMesh(core_axis_name="c", subcore_axis_name="s"),
    scratch_types=[
        pltpu.VMEM((_N, _D), jnp.float32),
        pltpu.VMEM((_N, _D), jnp.float32),
        pltpu.VMEM((_N, 2 * _D), jnp.float32),
        pltpu.VMEM((_N, 2 * _D), jnp.float32),
        pltpu.SemaphoreType.DMA,
        pltpu.SemaphoreType.DMA,
        pltpu.SemaphoreType.DMA,
        pltpu.SemaphoreType.DMA,
    ],
)
def _sc_encode(x_hbm, out_hbm, x_v0, x_v1, o_v0, o_v1, si0, si1, so0, so1):
    wid = lax.axis_index("s") * _NC + lax.axis_index("c")
    base = wid * _BPW
    xv, ov, sin, sout = [x_v0, x_v1], [o_v0, o_v1], [si0, si1], [so0, so1]
    cin = [None] * _BPW
    cout = [None] * _BPW
    cin[0] = pltpu.async_copy(x_hbm.at[base], xv[0], sin[0])
    cin[1] = pltpu.async_copy(x_hbm.at[base + 1], xv[1], sin[1])
    for bi in range(_BPW):
        p = bi % 2
        cin[bi].wait()
        if bi >= 2:
            cout[bi - 2].wait()
        _compute(xv[p], ov[p])
        cout[bi] = pltpu.async_copy(ov[p], out_hbm.at[base + bi], sout[p])
        if bi + 2 < _BPW:
            cin[bi + 2] = pltpu.async_copy(x_hbm.at[base + bi + 2], xv[p], sin[p])
    cout[_BPW - 2].wait()
    cout[_BPW - 1].wait()


def kernel(inputs, send_edges, recv_edges, edge2node_mat):
    return _sc_encode(inputs)


# speedup vs baseline: 6.0250x; 1.5243x over previous
import functools

import jax
import jax.numpy as jnp
from jax import lax
from jax.experimental import pallas as pl
from jax.experimental.pallas import tpu as pltpu
from jax.experimental.pallas import tpu_sc as plsc

_B, _N, _D = 128, 64, 128
_NC, _NS, _L = 2, 16, 16
_NW = _NC * _NS
_BPW = _B // _NW
_NG = _D // _L
_INV = 1.0 / (_N - 1)


def _compute(x_v, o_v, acc_v):
    zero = jnp.zeros((_L,), jnp.float32)
    for g in range(_NG):
        acc_v[pl.ds(_L * g, _L)] = zero

    @functools.partial(plsc.parallel_loop, 0, _N, unroll=4)
    def _acc(n):
        for g in range(_NG):
            plsc.addupdate(acc_v.at[pl.ds(_L * g, _L)], x_v[n, pl.ds(_L * g, _L)])

    accs = tuple(acc_v[pl.ds(_L * g, _L)] for g in range(_NG))

    @functools.partial(plsc.parallel_loop, 0, _N, unroll=4)
    def _emit(n):
        for g in range(_NG):
            xv = x_v[n, pl.ds(_L * g, _L)]
            o_v[n, pl.ds(_L * g, _L)] = (accs[g] - xv) * _INV
            o_v[n, pl.ds(_D + _L * g, _L)] = xv


@functools.partial(
    pl.kernel,
    out_type=jax.ShapeDtypeStruct((_B, _N, 2 * _D), jnp.float32),
    mesh=plsc.VectorSubcoreMesh(core_axis_name="c", subcore_axis_name="s"),
    scratch_types=[
        pltpu.VMEM((_N, _D), jnp.float32),
        pltpu.VMEM((_N, _D), jnp.float32),
        pltpu.VMEM((_N, 2 * _D), jnp.float32),
        pltpu.VMEM((_N, 2 * _D), jnp.float32),
        pltpu.VMEM((_D,), jnp.float32),
        pltpu.SemaphoreType.DMA,
        pltpu.SemaphoreType.DMA,
        pltpu.SemaphoreType.DMA,
        pltpu.SemaphoreType.DMA,
    ],
)
def _sc_encode(x_hbm, out_hbm, x_v0, x_v1, o_v0, o_v1, acc_v, si0, si1, so0, so1):
    wid = lax.axis_index("s") * _NC + lax.axis_index("c")
    base = wid * _BPW
    xv, ov, sin, sout = [x_v0, x_v1], [o_v0, o_v1], [si0, si1], [so0, so1]
    cin = [None] * _BPW
    cout = [None] * _BPW
    cin[0] = pltpu.async_copy(x_hbm.at[base], xv[0], sin[0])
    cin[1] = pltpu.async_copy(x_hbm.at[base + 1], xv[1], sin[1])
    for bi in range(_BPW):
        p = bi % 2
        cin[bi].wait()
        if bi >= 2:
            cout[bi - 2].wait()
        _compute(xv[p], ov[p], acc_v)
        cout[bi] = pltpu.async_copy(ov[p], out_hbm.at[base + bi], sout[p])
        if bi + 2 < _BPW:
            cin[bi + 2] = pltpu.async_copy(x_hbm.at[base + bi + 2], xv[p], sin[p])
    cout[_BPW - 2].wait()
    cout[_BPW - 1].wait()


def kernel(inputs, send_edges, recv_edges, edge2node_mat):
    return _sc_encode(inputs)
